# baseline (speedup 1.0000x reference)
import functools

import jax
import jax.numpy as jnp
from jax import lax
from jax.experimental import pallas as pl
from jax.experimental.pallas import tpu as pltpu
from jax.experimental.pallas import tpu_sc as plsc

F32 = jnp.float32
H = 128
B = 32
LD = 512
LQ = 32
DEMB = 300
DG = 384


def _sig(x):
    return 0.5 * jnp.tanh(0.5 * x) + 0.5


def _make_sc_gather(n_rows, d):
    info = plsc.get_sparse_core_info()
    nw = info.num_cores * info.num_subcores
    per_w = n_rows // nw
    chunk = per_w
    while chunk * d * 4 > 300_000 or per_w % chunk:
        chunk -= 8
    n_ch = per_w // chunk
    mesh = plsc.VectorSubcoreMesh(core_axis_name="c", subcore_axis_name="s")

    @functools.partial(
        pl.kernel,
        mesh=mesh,
        out_type=jax.ShapeDtypeStruct((n_rows, d), F32),
        scratch_types=[
            pltpu.VMEM((chunk,), jnp.int32),
            pltpu.VMEM((chunk, d), F32),
            pltpu.SemaphoreType.DMA,
            pltpu.VMEM((chunk,), jnp.int32),
            pltpu.VMEM((chunk, d), F32),
            pltpu.SemaphoreType.DMA,
        ],
    )
    def k(table_hbm, idx_hbm, out_hbm, idx0, rows0, sem0, idx1, rows1, sem1):
        wid = lax.axis_index("s") * info.num_cores + lax.axis_index("c")
        base = wid * per_w
        bufs = [(idx0, rows0, sem0), (idx1, rows1, sem1)]
        pend = [None, None]

        def start(c):
            ib, rb, sm = bufs[c % 2]
            pltpu.sync_copy(idx_hbm.at[pl.ds(base + c * chunk, chunk)], ib)
            pend[c % 2] = pltpu.async_copy(table_hbm.at[ib], rb, sm)

        start(0)
        for c in range(n_ch):
            if c + 1 < n_ch:
                start(c + 1)
            pend[c % 2].wait()
            pltpu.sync_copy(bufs[c % 2][1],
                            out_hbm.at[pl.ds(base + c * chunk, chunk)])

    return k


_TCH = 2048


def _tpad_body(x_ref, o_ref):
    xT = jnp.swapaxes(x_ref[...], 0, 1)
    o_ref[...] = jnp.concatenate(
        [xT, jnp.zeros((_TCH, DG - DEMB), F32)], axis=1)


def _transpose_pad(embT):
    nblk = (embT.shape[1] + _TCH - 1) // _TCH
    return pl.pallas_call(
        _tpad_body,
        grid=(nblk,),
        in_specs=[pl.BlockSpec((DEMB, _TCH), lambda i: (0, i))],
        out_specs=pl.BlockSpec((_TCH, DG), lambda i: (i, 0)),
        out_shape=jax.ShapeDtypeStruct((nblk * _TCH, DG), F32),
    )(embT)


def _layer_body(x1f_ref, x2f_ref, x1b_ref, x2b_ref,
                W1f_ref, W2f_ref, Whf_ref, bf_ref,
                W1b_ref, W2b_ref, Whb_ref, bb_ref,
                outf_ref, outb_ref,
                Gf_ref, Gb_ref, hf_ref, cf_ref, hb_ref, cb_ref,
                *, ch, nch):
    i = pl.program_id(0)
    d1 = x1f_ref.shape[2]
    d2 = x2f_ref.shape[2]

    X1f = x1f_ref[...].reshape(ch * B, d1)
    X2f = x2f_ref[...].reshape(ch * B, d2)
    X1b = x1b_ref[...].reshape(ch * B, d1)
    X2b = x2b_ref[...].reshape(ch * B, d2)
    W1f, W2f = W1f_ref[...], W2f_ref[...]
    W1b, W2b = W1b_ref[...], W2b_ref[...]
    bfv, bbv = bf_ref[...], bb_ref[...]

    @pl.when(i == 0)
    def _init():
        z = jnp.zeros((B, H), F32)
        hf_ref[...] = z
        cf_ref[...] = z
        hb_ref[...] = z
        cb_ref[...] = z

    hf, cf = hf_ref[...], cf_ref[...]
    hb, cb = hb_ref[...], cb_ref[...]
    Whf, Whb = Whf_ref[...], Whb_ref[...]

    sub = 8
    NP = ch // sub
    for k in range(NP):
        lo = k * sub * B
        Gf_ref[lo:lo + sub * B, :] = (
            jnp.dot(X1f[lo:lo + sub * B], W1f, preferred_element_type=F32)
            + jnp.dot(X2f[lo:lo + sub * B], W2f, preferred_element_type=F32)
            + bfv)
        blo = (NP - 1 - k) * sub * B
        Gb_ref[blo:blo + sub * B, :] = (
            jnp.dot(X1b[blo:blo + sub * B], W1b, preferred_element_type=F32)
            + jnp.dot(X2b[blo:blo + sub * B], W2b, preferred_element_type=F32)
            + bbv)
        for t in range(k * sub, (k + 1) * sub):
            gf = Gf_ref[t * B:(t + 1) * B, :] + jnp.dot(
                hf, Whf, preferred_element_type=F32)
            tb = ch - 1 - t
            gb = Gb_ref[tb * B:(tb + 1) * B, :] + jnp.dot(
                hb, Whb, preferred_element_type=F32)
            cf = _sig(gf[:, H:2 * H]) * cf + _sig(gf[:, :H]) * jnp.tanh(
                gf[:, 2 * H:3 * H])
            hf = _sig(gf[:, 3 * H:]) * jnp.tanh(cf)
            outf_ref[t] = hf
            cb = _sig(gb[:, H:2 * H]) * cb + _sig(gb[:, :H]) * jnp.tanh(
                gb[:, 2 * H:3 * H])
            hb = _sig(gb[:, 3 * H:]) * jnp.tanh(cb)
            outb_ref[tb] = hb

    hf_ref[...] = hf
    cf_ref[...] = cf
    hb_ref[...] = hb
    cb_ref[...] = cb


def _layer(x1, x2, W1f, W2f, Whf, bf, W1b, W2b, Whb, bb, ch=64):
    T = x1.shape[0]
    d1, d2 = x1.shape[2], x2.shape[2]
    nch = T // ch
    wspec = lambda shape: pl.BlockSpec(shape, lambda i: (0,) * len(shape))
    return pl.pallas_call(
        functools.partial(_layer_body, ch=ch, nch=nch),
        grid=(nch,),
        in_specs=[
            pl.BlockSpec((ch, B, d1), lambda i: (i, 0, 0)),
            pl.BlockSpec((ch, B, d2), lambda i: (i, 0, 0)),
            pl.BlockSpec((ch, B, d1), lambda i, n=nch: (n - 1 - i, 0, 0)),
            pl.BlockSpec((ch, B, d2), lambda i, n=nch: (n - 1 - i, 0, 0)),
            wspec((d1, 4 * H)), wspec((d2, 4 * H)), wspec((H, 4 * H)),
            wspec((1, 4 * H)),
            wspec((d1, 4 * H)), wspec((d2, 4 * H)), wspec((H, 4 * H)),
            wspec((1, 4 * H)),
        ],
        out_specs=[
            pl.BlockSpec((ch, B, H), lambda i: (i, 0, 0)),
            pl.BlockSpec((ch, B, H), lambda i, n=nch: (n - 1 - i, 0, 0)),
        ],
        out_shape=[
            jax.ShapeDtypeStruct((T, B, H), F32),
            jax.ShapeDtypeStruct((T, B, H), F32),
        ],
        scratch_shapes=[
            pltpu.VMEM((ch * B, 4 * H), F32),
            pltpu.VMEM((ch * B, 4 * H), F32),
            pltpu.VMEM((B, H), F32),
            pltpu.VMEM((B, H), F32),
            pltpu.VMEM((B, H), F32),
            pltpu.VMEM((B, H), F32),
        ],
    )(x1, x2, x1, x2, W1f, W2f, Whf, bf, W1b, W2b, Whb, bb)


def _q_body(xq_ref, mask_ref,
            W0f_ref, Wh0f_ref, b0f_ref, W0b_ref, Wh0b_ref, b0b_ref,
            W1ff_ref, W1fb_ref, Wh1f_ref, b1f_ref,
            W1bf_ref, W1bb_ref, Wh1b_ref, b1b_ref,
            wq_ref, WsT_ref, WeT_ref,
            vs_ref, ve_ref,
            Gf_ref, Gb_ref, H0f_ref, H0b_ref, H1f_ref, H1b_ref):
    X = xq_ref[...].reshape(LQ * B, DG)

    def birun(Whf, Whb, Hf_ref, Hb_ref):
        hf = jnp.zeros((B, H), F32)
        cf = jnp.zeros((B, H), F32)
        hb = jnp.zeros((B, H), F32)
        cb = jnp.zeros((B, H), F32)
        for t in range(LQ):
            tb = LQ - 1 - t
            gf = Gf_ref[t * B:(t + 1) * B, :] + jnp.dot(
                hf, Whf, preferred_element_type=F32)
            gb = Gb_ref[tb * B:(tb + 1) * B, :] + jnp.dot(
                hb, Whb, preferred_element_type=F32)
            cf = _sig(gf[:, H:2 * H]) * cf + _sig(gf[:, :H]) * jnp.tanh(
                gf[:, 2 * H:3 * H])
            hf = _sig(gf[:, 3 * H:]) * jnp.tanh(cf)
            Hf_ref[t * B:(t + 1) * B, :] = hf
            cb = _sig(gb[:, H:2 * H]) * cb + _sig(gb[:, :H]) * jnp.tanh(
                gb[:, 2 * H:3 * H])
            hb = _sig(gb[:, 3 * H:]) * jnp.tanh(cb)
            Hb_ref[tb * B:(tb + 1) * B, :] = hb

    Gf_ref[...] = jnp.dot(X, W0f_ref[...], preferred_element_type=F32) + b0f_ref[...]
    Gb_ref[...] = jnp.dot(X, W0b_ref[...], preferred_element_type=F32) + b0b_ref[...]
    birun(Wh0f_ref[...], Wh0b_ref[...], H0f_ref, H0b_ref)
    h0f = H0f_ref[...]
    h0b = H0b_ref[...]
    Gf_ref[...] = (jnp.dot(h0f, W1ff_ref[...], preferred_element_type=F32)
                   + jnp.dot(h0b, W1fb_ref[...], preferred_element_type=F32)
                   + b1f_ref[...])
    Gb_ref[...] = (jnp.dot(h0f, W1bf_ref[...], preferred_element_type=F32)
                   + jnp.dot(h0b, W1bb_ref[...], preferred_element_type=F32)
                   + b1b_ref[...])
    birun(Wh1f_ref[...], Wh1b_ref[...], H1f_ref, H1b_ref)

    qf3 = H1f_ref[...].reshape(LQ, B, H)
    qb3 = H1b_ref[...].reshape(LQ, B, H)
    wq = wq_ref[...]
    scores = (jnp.sum(qf3 * wq[0, :H].reshape(1, 1, H), axis=-1)
              + jnp.sum(qb3 * wq[0, H:].reshape(1, 1, H), axis=-1))
    maskq = jnp.swapaxes(mask_ref[...].astype(F32), 0, 1)
    scores = jnp.where(maskq > 0.0, -1e30, scores)
    m = jnp.max(scores, axis=0, keepdims=True)
    e = jnp.exp(scores - m)
    alpha = e / jnp.sum(e, axis=0, keepdims=True)
    qm_f = jnp.sum(alpha[:, :, None] * qf3, axis=0)
    qm_b = jnp.sum(alpha[:, :, None] * qb3, axis=0)
    vs_ref[...] = (jnp.dot(qm_f, WsT_ref[:H], preferred_element_type=F32)
                   + jnp.dot(qm_b, WsT_ref[H:], preferred_element_type=F32))
    ve_ref[...] = (jnp.dot(qm_f, WeT_ref[:H], preferred_element_type=F32)
                   + jnp.dot(qm_b, WeT_ref[H:], preferred_element_type=F32))


def _qblock(xq, maskq, W0f, Wh0f, b0f, W0b, Wh0b, b0b,
            W1ff, W1fb, Wh1f, b1f, W1bf, W1bb, Wh1b, b1b, wq, WsT, WeT):
    return pl.pallas_call(
        _q_body,
        out_shape=[
            jax.ShapeDtypeStruct((B, 2 * H), F32),
            jax.ShapeDtypeStruct((B, 2 * H), F32),
        ],
        scratch_shapes=[
            pltpu.VMEM((LQ * B, 4 * H), F32),
            pltpu.VMEM((LQ * B, 4 * H), F32),
            pltpu.VMEM((LQ * B, H), F32),
            pltpu.VMEM((LQ * B, H), F32),
            pltpu.VMEM((LQ * B, H), F32),
            pltpu.VMEM((LQ * B, H), F32),
        ],
    )(xq, maskq, W0f, Wh0f, b0f, W0b, Wh0b, b0b,
      W1ff, W1fb, Wh1f, b1f, W1bf, W1bb, Wh1b, b1b, wq, WsT, WeT)


def _final_body(hf_ref, hb_ref, vs_ref, ve_ref, mask_ref, out_ref, S_ref, E_ref):
    vs = vs_ref[...]
    ve = ve_ref[...]
    vsf = vs[:, :H].reshape(1, B, H)
    vsb = vs[:, H:].reshape(1, B, H)
    vef = ve[:, :H].reshape(1, B, H)
    veb = ve[:, H:].reshape(1, B, H)
    CH = 64
    for c in range(LD // CH):
        sl = pl.ds(c * CH, CH)
        hf = hf_ref[sl]
        hb = hb_ref[sl]
        S_ref[sl] = jnp.sum(hf * vsf, axis=-1) + jnp.sum(hb * vsb, axis=-1)
        E_ref[sl] = jnp.sum(hf * vef, axis=-1) + jnp.sum(hb * veb, axis=-1)

    mask = mask_ref[...]

    def logsm(x_tb):
        x = jnp.where(mask, -1e30, jnp.swapaxes(x_tb, 0, 1))
        m = jnp.max(x, axis=1, keepdims=True)
        return x - m - jnp.log(jnp.sum(jnp.exp(x - m), axis=1, keepdims=True))

    out_ref[0] = logsm(S_ref[...])
    out_ref[1] = logsm(E_ref[...])


def _final(hf1, hb1, vs, ve, maskd):
    return pl.pallas_call(
        _final_body,
        out_shape=jax.ShapeDtypeStruct((2, B, LD), F32),
        scratch_shapes=[
            pltpu.VMEM((LD, B), F32),
            pltpu.VMEM((LD, B), F32),
        ],
    )(hf1, hb1, vs, ve, maskd)


def kernel(x1, x1_f, x1_mask, x2, x2_mask, emb,
           d0_Wxf, d0_Whf, d0_bf, d0_Wxb, d0_Whb, d0_bb,
           d1_Wxf, d1_Whf, d1_bf, d1_Wxb, d1_Whb, d1_bb,
           q0_Wxf, q0_Whf, q0_bf, q0_Wxb, q0_Whb, q0_bb,
           q1_Wxf, q1_Whf, q1_bf, q1_Wxb, q1_Whb, q1_bb,
           wq, Ws, We):
    embp = _transpose_pad(jnp.swapaxes(emb, 0, 1))
    xd = _make_sc_gather(LD * B, DG)(
        embp, x1.T.reshape(-1).astype(jnp.int32)).reshape(LD, B, DG)
    xq = _make_sc_gather(LQ * B, DG)(
        embp, x2.T.reshape(-1).astype(jnp.int32)).reshape(LQ, B, DG)
    fd = jnp.transpose(x1_f, (1, 0, 2))

    b2 = lambda b: b.reshape(1, 4 * H)
    wpad = lambda W: jnp.pad(W[:DEMB], ((0, DG - DEMB), (0, 0)))
    hf0, hb0 = _layer(xd, fd,
                      wpad(d0_Wxf), d0_Wxf[DEMB:], d0_Whf, b2(d0_bf),
                      wpad(d0_Wxb), d0_Wxb[DEMB:], d0_Whb, b2(d0_bb))
    hf1, hb1 = _layer(hf0, hb0,
                      d1_Wxf[:H], d1_Wxf[H:], d1_Whf, b2(d1_bf),
                      d1_Wxb[:H], d1_Wxb[H:], d1_Whb, b2(d1_bb), ch=128)

    vs, ve = _qblock(xq, x2_mask,
                     wpad(q0_Wxf), q0_Whf, b2(q0_bf),
                     wpad(q0_Wxb), q0_Whb, b2(q0_bb),
                     q1_Wxf[:H], q1_Wxf[H:], q1_Whf, b2(q1_bf),
                     q1_Wxb[:H], q1_Wxb[H:], q1_Whb, b2(q1_bb),
                     wq.reshape(1, 2 * H), Ws.T, We.T)

    return _final(hf1, hb1, vs, ve, x1_mask)


# speedup vs baseline: 1.0288x; 1.0288x over previous
import functools

import jax
import jax.numpy as jnp
from jax import lax
from jax.experimental import pallas as pl
from jax.experimental.pallas import tpu as pltpu
from jax.experimental.pallas import tpu_sc as plsc

F32 = jnp.float32
H = 128
B = 32
LD = 512
LQ = 32
DEMB = 300
DG = 384


def _sig(x):
    return 0.5 * jnp.tanh(0.5 * x) + 0.5


def _make_sc_gather(n_rows, d):
    info = plsc.get_sparse_core_info()
    nw = info.num_cores * info.num_subcores
    per_w = n_rows // nw
    chunk = per_w
    while chunk * d * 4 > 300_000 or per_w % chunk:
        chunk -= 8
    n_ch = per_w // chunk
    mesh = plsc.VectorSubcoreMesh(core_axis_name="c", subcore_axis_name="s")

    @functools.partial(
        pl.kernel,
        mesh=mesh,
        out_type=jax.ShapeDtypeStruct((n_rows, d), F32),
        scratch_types=[
            pltpu.VMEM((chunk,), jnp.int32),
            pltpu.VMEM((chunk, d), F32),
            pltpu.SemaphoreType.DMA,
            pltpu.VMEM((chunk,), jnp.int32),
            pltpu.VMEM((chunk, d), F32),
            pltpu.SemaphoreType.DMA,
        ],
    )
    def k(table_hbm, idx_hbm, out_hbm, idx0, rows0, sem0, idx1, rows1, sem1):
        wid = lax.axis_index("s") * info.num_cores + lax.axis_index("c")
        base = wid * per_w
        bufs = [(idx0, rows0, sem0), (idx1, rows1, sem1)]
        pend = [None, None]

        def start(c):
            ib, rb, sm = bufs[c % 2]
            pltpu.sync_copy(idx_hbm.at[pl.ds(base + c * chunk, chunk)], ib)
            pend[c % 2] = pltpu.async_copy(table_hbm.at[ib], rb, sm)

        start(0)
        for c in range(n_ch):
            if c + 1 < n_ch:
                start(c + 1)
            pend[c % 2].wait()
            pltpu.sync_copy(bufs[c % 2][1],
                            out_hbm.at[pl.ds(base + c * chunk, chunk)])

    return k


_TCH = 2048


def _tpad_body(x_ref, o_ref):
    xT = jnp.swapaxes(x_ref[...], 0, 1)
    o_ref[...] = jnp.concatenate(
        [xT, jnp.zeros((_TCH, DG - DEMB), F32)], axis=1)


def _transpose_pad(embT):
    nblk = (embT.shape[1] + _TCH - 1) // _TCH
    return pl.pallas_call(
        _tpad_body,
        grid=(nblk,),
        in_specs=[pl.BlockSpec((DEMB, _TCH), lambda i: (0, i))],
        out_specs=pl.BlockSpec((_TCH, DG), lambda i: (i, 0)),
        out_shape=jax.ShapeDtypeStruct((nblk * _TCH, DG), F32),
    )(embT)


def _layer_body(x1f_ref, x2f_ref, x1b_ref, x2b_ref,
                W1f_ref, W2f_ref, Whf_ref, bf_ref,
                W1b_ref, W2b_ref, Whb_ref, bb_ref,
                outf_ref, outb_ref,
                Gf_ref, Gb_ref, hf_ref, cf_ref, hb_ref, cb_ref,
                *, ch, nch):
    i = pl.program_id(0)
    d1 = x1f_ref.shape[2]
    d2 = x2f_ref.shape[2]

    X1f = x1f_ref[...].reshape(ch * B, d1)
    X2f = x2f_ref[...].reshape(ch * B, d2)
    X1b = x1b_ref[...].reshape(ch * B, d1)
    X2b = x2b_ref[...].reshape(ch * B, d2)
    W1f, W2f = W1f_ref[...], W2f_ref[...]
    W1b, W2b = W1b_ref[...], W2b_ref[...]
    bfv, bbv = bf_ref[...], bb_ref[...]

    @pl.when(i == 0)
    def _init():
        z = jnp.zeros((B, H), F32)
        hf_ref[...] = z
        cf_ref[...] = z
        hb_ref[...] = z
        cb_ref[...] = z

    hf, cf = hf_ref[...], cf_ref[...]
    hb, cb = hb_ref[...], cb_ref[...]
    Whf, Whb = Whf_ref[...], Whb_ref[...]

    sub = 4
    NP = ch // sub
    for k in range(NP):
        lo = k * sub * B
        Gf_ref[lo:lo + sub * B, :] = (
            jnp.dot(X1f[lo:lo + sub * B], W1f, preferred_element_type=F32)
            + jnp.dot(X2f[lo:lo + sub * B], W2f, preferred_element_type=F32)
            + bfv)
        blo = (NP - 1 - k) * sub * B
        Gb_ref[blo:blo + sub * B, :] = (
            jnp.dot(X1b[blo:blo + sub * B], W1b, preferred_element_type=F32)
            + jnp.dot(X2b[blo:blo + sub * B], W2b, preferred_element_type=F32)
            + bbv)
        for t in range(k * sub, (k + 1) * sub):
            gf = Gf_ref[t * B:(t + 1) * B, :] + jnp.dot(
                hf, Whf, preferred_element_type=F32)
            tb = ch - 1 - t
            gb = Gb_ref[tb * B:(tb + 1) * B, :] + jnp.dot(
                hb, Whb, preferred_element_type=F32)
            cf = _sig(gf[:, H:2 * H]) * cf + _sig(gf[:, :H]) * jnp.tanh(
                gf[:, 2 * H:3 * H])
            hf = _sig(gf[:, 3 * H:]) * jnp.tanh(cf)
            outf_ref[t] = hf
            cb = _sig(gb[:, H:2 * H]) * cb + _sig(gb[:, :H]) * jnp.tanh(
                gb[:, 2 * H:3 * H])
            hb = _sig(gb[:, 3 * H:]) * jnp.tanh(cb)
            outb_ref[tb] = hb

    hf_ref[...] = hf
    cf_ref[...] = cf
    hb_ref[...] = hb
    cb_ref[...] = cb


def _layer(x1, x2, W1f, W2f, Whf, bf, W1b, W2b, Whb, bb, ch=64):
    T = x1.shape[0]
    d1, d2 = x1.shape[2], x2.shape[2]
    nch = T // ch
    wspec = lambda shape: pl.BlockSpec(shape, lambda i: (0,) * len(shape))
    return pl.pallas_call(
        functools.partial(_layer_body, ch=ch, nch=nch),
        grid=(nch,),
        in_specs=[
            pl.BlockSpec((ch, B, d1), lambda i: (i, 0, 0)),
            pl.BlockSpec((ch, B, d2), lambda i: (i, 0, 0)),
            pl.BlockSpec((ch, B, d1), lambda i, n=nch: (n - 1 - i, 0, 0)),
            pl.BlockSpec((ch, B, d2), lambda i, n=nch: (n - 1 - i, 0, 0)),
            wspec((d1, 4 * H)), wspec((d2, 4 * H)), wspec((H, 4 * H)),
            wspec((1, 4 * H)),
            wspec((d1, 4 * H)), wspec((d2, 4 * H)), wspec((H, 4 * H)),
            wspec((1, 4 * H)),
        ],
        out_specs=[
            pl.BlockSpec((ch, B, H), lambda i: (i, 0, 0)),
            pl.BlockSpec((ch, B, H), lambda i, n=nch: (n - 1 - i, 0, 0)),
        ],
        out_shape=[
            jax.ShapeDtypeStruct((T, B, H), F32),
            jax.ShapeDtypeStruct((T, B, H), F32),
        ],
        scratch_shapes=[
            pltpu.VMEM((ch * B, 4 * H), F32),
            pltpu.VMEM((ch * B, 4 * H), F32),
            pltpu.VMEM((B, H), F32),
            pltpu.VMEM((B, H), F32),
            pltpu.VMEM((B, H), F32),
            pltpu.VMEM((B, H), F32),
        ],
    )(x1, x2, x1, x2, W1f, W2f, Whf, bf, W1b, W2b, Whb, bb)


def _q_body(xq_ref, mask_ref,
            W0f_ref, Wh0f_ref, b0f_ref, W0b_ref, Wh0b_ref, b0b_ref,
            W1ff_ref, W1fb_ref, Wh1f_ref, b1f_ref,
            W1bf_ref, W1bb_ref, Wh1b_ref, b1b_ref,
            wq_ref, WsT_ref, WeT_ref,
            vs_ref, ve_ref,
            Gf_ref, Gb_ref, H0f_ref, H0b_ref, H1f_ref, H1b_ref):
    X = xq_ref[...].reshape(LQ * B, DG)

    def birun(Whf, Whb, Hf_ref, Hb_ref):
        hf = jnp.zeros((B, H), F32)
        cf = jnp.zeros((B, H), F32)
        hb = jnp.zeros((B, H), F32)
        cb = jnp.zeros((B, H), F32)
        for t in range(LQ):
            tb = LQ - 1 - t
            gf = Gf_ref[t * B:(t + 1) * B, :] + jnp.dot(
                hf, Whf, preferred_element_type=F32)
            gb = Gb_ref[tb * B:(tb + 1) * B, :] + jnp.dot(
                hb, Whb, preferred_element_type=F32)
            cf = _sig(gf[:, H:2 * H]) * cf + _sig(gf[:, :H]) * jnp.tanh(
                gf[:, 2 * H:3 * H])
            hf = _sig(gf[:, 3 * H:]) * jnp.tanh(cf)
            Hf_ref[t * B:(t + 1) * B, :] = hf
            cb = _sig(gb[:, H:2 * H]) * cb + _sig(gb[:, :H]) * jnp.tanh(
                gb[:, 2 * H:3 * H])
            hb = _sig(gb[:, 3 * H:]) * jnp.tanh(cb)
            Hb_ref[tb * B:(tb + 1) * B, :] = hb

    Gf_ref[...] = jnp.dot(X, W0f_ref[...], preferred_element_type=F32) + b0f_ref[...]
    Gb_ref[...] = jnp.dot(X, W0b_ref[...], preferred_element_type=F32) + b0b_ref[...]
    birun(Wh0f_ref[...], Wh0b_ref[...], H0f_ref, H0b_ref)
    h0f = H0f_ref[...]
    h0b = H0b_ref[...]
    Gf_ref[...] = (jnp.dot(h0f, W1ff_ref[...], preferred_element_type=F32)
                   + jnp.dot(h0b, W1fb_ref[...], preferred_element_type=F32)
                   + b1f_ref[...])
    Gb_ref[...] = (jnp.dot(h0f, W1bf_ref[...], preferred_element_type=F32)
                   + jnp.dot(h0b, W1bb_ref[...], preferred_element_type=F32)
                   + b1b_ref[...])
    birun(Wh1f_ref[...], Wh1b_ref[...], H1f_ref, H1b_ref)

    qf3 = H1f_ref[...].reshape(LQ, B, H)
    qb3 = H1b_ref[...].reshape(LQ, B, H)
    wq = wq_ref[...]
    scores = (jnp.sum(qf3 * wq[0, :H].reshape(1, 1, H), axis=-1)
              + jnp.sum(qb3 * wq[0, H:].reshape(1, 1, H), axis=-1))
    maskq = jnp.swapaxes(mask_ref[...].astype(F32), 0, 1)
    scores = jnp.where(maskq > 0.0, -1e30, scores)
    m = jnp.max(scores, axis=0, keepdims=True)
    e = jnp.exp(scores - m)
    alpha = e / jnp.sum(e, axis=0, keepdims=True)
    qm_f = jnp.sum(alpha[:, :, None] * qf3, axis=0)
    qm_b = jnp.sum(alpha[:, :, None] * qb3, axis=0)
    vs_ref[...] = (jnp.dot(qm_f, WsT_ref[:H], preferred_element_type=F32)
                   + jnp.dot(qm_b, WsT_ref[H:], preferred_element_type=F32))
    ve_ref[...] = (jnp.dot(qm_f, WeT_ref[:H], preferred_element_type=F32)
                   + jnp.dot(qm_b, WeT_ref[H:], preferred_element_type=F32))


def _qblock(xq, maskq, W0f, Wh0f, b0f, W0b, Wh0b, b0b,
            W1ff, W1fb, Wh1f, b1f, W1bf, W1bb, Wh1b, b1b, wq, WsT, WeT):
    return pl.pallas_call(
        _q_body,
        out_shape=[
            jax.ShapeDtypeStruct((B, 2 * H), F32),
            jax.ShapeDtypeStruct((B, 2 * H), F32),
        ],
        scratch_shapes=[
            pltpu.VMEM((LQ * B, 4 * H), F32),
            pltpu.VMEM((LQ * B, 4 * H), F32),
            pltpu.VMEM((LQ * B, H), F32),
            pltpu.VMEM((LQ * B, H), F32),
            pltpu.VMEM((LQ * B, H), F32),
            pltpu.VMEM((LQ * B, H), F32),
        ],
    )(xq, maskq, W0f, Wh0f, b0f, W0b, Wh0b, b0b,
      W1ff, W1fb, Wh1f, b1f, W1bf, W1bb, Wh1b, b1b, wq, WsT, WeT)


def _final_body(hf_ref, hb_ref, vs_ref, ve_ref, mask_ref, out_ref, S_ref, E_ref):
    vs = vs_ref[...]
    ve = ve_ref[...]
    vsf = vs[:, :H].reshape(1, B, H)
    vsb = vs[:, H:].reshape(1, B, H)
    vef = ve[:, :H].reshape(1, B, H)
    veb = ve[:, H:].reshape(1, B, H)
    CH = 64
    for c in range(LD // CH):
        sl = pl.ds(c * CH, CH)
        hf = hf_ref[sl]
        hb = hb_ref[sl]
        S_ref[sl] = jnp.sum(hf * vsf, axis=-1) + jnp.sum(hb * vsb, axis=-1)
        E_ref[sl] = jnp.sum(hf * vef, axis=-1) + jnp.sum(hb * veb, axis=-1)

    mask = mask_ref[...]

    def logsm(x_tb):
        x = jnp.where(mask, -1e30, jnp.swapaxes(x_tb, 0, 1))
        m = jnp.max(x, axis=1, keepdims=True)
        return x - m - jnp.log(jnp.sum(jnp.exp(x - m), axis=1, keepdims=True))

    out_ref[0] = logsm(S_ref[...])
    out_ref[1] = logsm(E_ref[...])


def _final(hf1, hb1, vs, ve, maskd):
    return pl.pallas_call(
        _final_body,
        out_shape=jax.ShapeDtypeStruct((2, B, LD), F32),
        scratch_shapes=[
            pltpu.VMEM((LD, B), F32),
            pltpu.VMEM((LD, B), F32),
        ],
    )(hf1, hb1, vs, ve, maskd)


def kernel(x1, x1_f, x1_mask, x2, x2_mask, emb,
           d0_Wxf, d0_Whf, d0_bf, d0_Wxb, d0_Whb, d0_bb,
           d1_Wxf, d1_Whf, d1_bf, d1_Wxb, d1_Whb, d1_bb,
           q0_Wxf, q0_Whf, q0_bf, q0_Wxb, q0_Whb, q0_bb,
           q1_Wxf, q1_Whf, q1_bf, q1_Wxb, q1_Whb, q1_bb,
           wq, Ws, We):
    embp = _transpose_pad(jnp.swapaxes(emb, 0, 1))
    xd = _make_sc_gather(LD * B, DG)(
        embp, x1.T.reshape(-1).astype(jnp.int32)).reshape(LD, B, DG)
    xq = _make_sc_gather(LQ * B, DG)(
        embp, x2.T.reshape(-1).astype(jnp.int32)).reshape(LQ, B, DG)
    fd = jnp.transpose(x1_f, (1, 0, 2))

    b2 = lambda b: b.reshape(1, 4 * H)
    wpad = lambda W: jnp.pad(W[:DEMB], ((0, DG - DEMB), (0, 0)))
    hf0, hb0 = _layer(xd, fd,
                      wpad(d0_Wxf), d0_Wxf[DEMB:], d0_Whf, b2(d0_bf),
                      wpad(d0_Wxb), d0_Wxb[DEMB:], d0_Whb, b2(d0_bb))
    hf1, hb1 = _layer(hf0, hb0,
                      d1_Wxf[:H], d1_Wxf[H:], d1_Whf, b2(d1_bf),
                      d1_Wxb[:H], d1_Wxb[H:], d1_Whb, b2(d1_bb))

    vs, ve = _qblock(xq, x2_mask,
                     wpad(q0_Wxf), q0_Whf, b2(q0_bf),
                     wpad(q0_Wxb), q0_Whb, b2(q0_bb),
                     q1_Wxf[:H], q1_Wxf[H:], q1_Whf, b2(q1_bf),
                     q1_Wxb[:H], q1_Wxb[H:], q1_Whb, b2(q1_bb),
                     wq.reshape(1, 2 * H), Ws.T, We.T)

    return _final(hf1, hb1, vs, ve, x1_mask)


# speedup vs baseline: 1.0324x; 1.0035x over previous
import functools

import jax
import jax.numpy as jnp
from jax import lax
from jax.experimental import pallas as pl
from jax.experimental.pallas import tpu as pltpu
from jax.experimental.pallas import tpu_sc as plsc

F32 = jnp.float32
H = 128
B = 32
LD = 512
LQ = 32
DEMB = 300
DG = 384


def _sig(x):
    return 0.5 * jnp.tanh(0.5 * x) + 0.5


def _make_sc_gather(n_rows, d):
    info = plsc.get_sparse_core_info()
    nw = info.num_cores * info.num_subcores
    per_w = n_rows // nw
    chunk = per_w
    while chunk * d * 4 > 300_000 or per_w % chunk:
        chunk -= 8
    n_ch = per_w // chunk
    mesh = plsc.VectorSubcoreMesh(core_axis_name="c", subcore_axis_name="s")

    @functools.partial(
        pl.kernel,
        mesh=mesh,
        out_type=jax.ShapeDtypeStruct((n_rows, d), F32),
        scratch_types=[
            pltpu.VMEM((chunk,), jnp.int32),
            pltpu.VMEM((chunk, d), F32),
            pltpu.SemaphoreType.DMA,
            pltpu.VMEM((chunk,), jnp.int32),
            pltpu.VMEM((chunk, d), F32),
            pltpu.SemaphoreType.DMA,
        ],
    )
    def k(table_hbm, idx_hbm, out_hbm, idx0, rows0, sem0, idx1, rows1, sem1):
        wid = lax.axis_index("s") * info.num_cores + lax.axis_index("c")
        base = wid * per_w
        bufs = [(idx0, rows0, sem0), (idx1, rows1, sem1)]
        pend = [None, None]

        def start(c):
            ib, rb, sm = bufs[c % 2]
            pltpu.sync_copy(idx_hbm.at[pl.ds(base + c * chunk, chunk)], ib)
            pend[c % 2] = pltpu.async_copy(table_hbm.at[ib], rb, sm)

        start(0)
        for c in range(n_ch):
            if c + 1 < n_ch:
                start(c + 1)
            pend[c % 2].wait()
            pltpu.sync_copy(bufs[c % 2][1],
                            out_hbm.at[pl.ds(base + c * chunk, chunk)])

    return k


_TCH = 2048


def _tpad_body(x_ref, o_ref):
    xT = jnp.swapaxes(x_ref[...], 0, 1)
    o_ref[...] = jnp.concatenate(
        [xT, jnp.zeros((_TCH, DG - DEMB), F32)], axis=1)


def _transpose_pad(embT):
    nblk = (embT.shape[1] + _TCH - 1) // _TCH
    return pl.pallas_call(
        _tpad_body,
        grid=(nblk,),
        in_specs=[pl.BlockSpec((DEMB, _TCH), lambda i: (0, i))],
        out_specs=pl.BlockSpec((_TCH, DG), lambda i: (i, 0)),
        out_shape=jax.ShapeDtypeStruct((nblk * _TCH, DG), F32),
    )(embT)


def _layer_body(x1f_ref, x2f_ref, x1b_ref, x2b_ref,
                W1f_ref, W2f_ref, Whf_ref, bf_ref,
                W1b_ref, W2b_ref, Whb_ref, bb_ref,
                outf_ref, outb_ref,
                Gf_ref, Gb_ref, hf_ref, cf_ref, hb_ref, cb_ref,
                *, ch, nch):
    i = pl.program_id(0)
    d1 = x1f_ref.shape[2]
    d2 = x2f_ref.shape[2]

    X1f = x1f_ref[...].reshape(ch * B, d1)
    X2f = x2f_ref[...].reshape(ch * B, d2)
    X1b = x1b_ref[...].reshape(ch * B, d1)
    X2b = x2b_ref[...].reshape(ch * B, d2)
    W1f, W2f = W1f_ref[...], W2f_ref[...]
    W1b, W2b = W1b_ref[...], W2b_ref[...]
    bfv, bbv = bf_ref[...], bb_ref[...]

    @pl.when(i == 0)
    def _init():
        z = jnp.zeros((B, H), F32)
        hf_ref[...] = z
        cf_ref[...] = z
        hb_ref[...] = z
        cb_ref[...] = z

    hf, cf = hf_ref[...], cf_ref[...]
    hb, cb = hb_ref[...], cb_ref[...]
    Whf, Whb = Whf_ref[...], Whb_ref[...]

    sub = 2
    NP = ch // sub
    for k in range(NP):
        lo = k * sub * B
        Gf_ref[lo:lo + sub * B, :] = (
            jnp.dot(X1f[lo:lo + sub * B], W1f, preferred_element_type=F32)
            + jnp.dot(X2f[lo:lo + sub * B], W2f, preferred_element_type=F32)
            + bfv)
        blo = (NP - 1 - k) * sub * B
        Gb_ref[blo:blo + sub * B, :] = (
            jnp.dot(X1b[blo:blo + sub * B], W1b, preferred_element_type=F32)
            + jnp.dot(X2b[blo:blo + sub * B], W2b, preferred_element_type=F32)
            + bbv)
        for t in range(k * sub, (k + 1) * sub):
            gf = Gf_ref[t * B:(t + 1) * B, :] + jnp.dot(
                hf, Whf, preferred_element_type=F32)
            tb = ch - 1 - t
            gb = Gb_ref[tb * B:(tb + 1) * B, :] + jnp.dot(
                hb, Whb, preferred_element_type=F32)
            cf = _sig(gf[:, H:2 * H]) * cf + _sig(gf[:, :H]) * jnp.tanh(
                gf[:, 2 * H:3 * H])
            hf = _sig(gf[:, 3 * H:]) * jnp.tanh(cf)
            outf_ref[t] = hf
            cb = _sig(gb[:, H:2 * H]) * cb + _sig(gb[:, :H]) * jnp.tanh(
                gb[:, 2 * H:3 * H])
            hb = _sig(gb[:, 3 * H:]) * jnp.tanh(cb)
            outb_ref[tb] = hb

    hf_ref[...] = hf
    cf_ref[...] = cf
    hb_ref[...] = hb
    cb_ref[...] = cb


def _layer(x1, x2, W1f, W2f, Whf, bf, W1b, W2b, Whb, bb, ch=64):
    T = x1.shape[0]
    d1, d2 = x1.shape[2], x2.shape[2]
    nch = T // ch
    wspec = lambda shape: pl.BlockSpec(shape, lambda i: (0,) * len(shape))
    return pl.pallas_call(
        functools.partial(_layer_body, ch=ch, nch=nch),
        grid=(nch,),
        in_specs=[
            pl.BlockSpec((ch, B, d1), lambda i: (i, 0, 0)),
            pl.BlockSpec((ch, B, d2), lambda i: (i, 0, 0)),
            pl.BlockSpec((ch, B, d1), lambda i, n=nch: (n - 1 - i, 0, 0)),
            pl.BlockSpec((ch, B, d2), lambda i, n=nch: (n - 1 - i, 0, 0)),
            wspec((d1, 4 * H)), wspec((d2, 4 * H)), wspec((H, 4 * H)),
            wspec((1, 4 * H)),
            wspec((d1, 4 * H)), wspec((d2, 4 * H)), wspec((H, 4 * H)),
            wspec((1, 4 * H)),
        ],
        out_specs=[
            pl.BlockSpec((ch, B, H), lambda i: (i, 0, 0)),
            pl.BlockSpec((ch, B, H), lambda i, n=nch: (n - 1 - i, 0, 0)),
        ],
        out_shape=[
            jax.ShapeDtypeStruct((T, B, H), F32),
            jax.ShapeDtypeStruct((T, B, H), F32),
        ],
        scratch_shapes=[
            pltpu.VMEM((ch * B, 4 * H), F32),
            pltpu.VMEM((ch * B, 4 * H), F32),
            pltpu.VMEM((B, H), F32),
            pltpu.VMEM((B, H), F32),
            pltpu.VMEM((B, H), F32),
            pltpu.VMEM((B, H), F32),
        ],
    )(x1, x2, x1, x2, W1f, W2f, Whf, bf, W1b, W2b, Whb, bb)


def _q_body(xq_ref, mask_ref,
            W0f_ref, Wh0f_ref, b0f_ref, W0b_ref, Wh0b_ref, b0b_ref,
            W1ff_ref, W1fb_ref, Wh1f_ref, b1f_ref,
            W1bf_ref, W1bb_ref, Wh1b_ref, b1b_ref,
            wq_ref, WsT_ref, WeT_ref,
            vs_ref, ve_ref,
            Gf_ref, Gb_ref, H0f_ref, H0b_ref, H1f_ref, H1b_ref):
    X = xq_ref[...].reshape(LQ * B, DG)

    def birun(Whf, Whb, Hf_ref, Hb_ref):
        hf = jnp.zeros((B, H), F32)
        cf = jnp.zeros((B, H), F32)
        hb = jnp.zeros((B, H), F32)
        cb = jnp.zeros((B, H), F32)
        for t in range(LQ):
            tb = LQ - 1 - t
            gf = Gf_ref[t * B:(t + 1) * B, :] + jnp.dot(
                hf, Whf, preferred_element_type=F32)
            gb = Gb_ref[tb * B:(tb + 1) * B, :] + jnp.dot(
                hb, Whb, preferred_element_type=F32)
            cf = _sig(gf[:, H:2 * H]) * cf + _sig(gf[:, :H]) * jnp.tanh(
                gf[:, 2 * H:3 * H])
            hf = _sig(gf[:, 3 * H:]) * jnp.tanh(cf)
            Hf_ref[t * B:(t + 1) * B, :] = hf
            cb = _sig(gb[:, H:2 * H]) * cb + _sig(gb[:, :H]) * jnp.tanh(
                gb[:, 2 * H:3 * H])
            hb = _sig(gb[:, 3 * H:]) * jnp.tanh(cb)
            Hb_ref[tb * B:(tb + 1) * B, :] = hb

    Gf_ref[...] = jnp.dot(X, W0f_ref[...], preferred_element_type=F32) + b0f_ref[...]
    Gb_ref[...] = jnp.dot(X, W0b_ref[...], preferred_element_type=F32) + b0b_ref[...]
    birun(Wh0f_ref[...], Wh0b_ref[...], H0f_ref, H0b_ref)
    h0f = H0f_ref[...]
    h0b = H0b_ref[...]
    Gf_ref[...] = (jnp.dot(h0f, W1ff_ref[...], preferred_element_type=F32)
                   + jnp.dot(h0b, W1fb_ref[...], preferred_element_type=F32)
                   + b1f_ref[...])
    Gb_ref[...] = (jnp.dot(h0f, W1bf_ref[...], preferred_element_type=F32)
                   + jnp.dot(h0b, W1bb_ref[...], preferred_element_type=F32)
                   + b1b_ref[...])
    birun(Wh1f_ref[...], Wh1b_ref[...], H1f_ref, H1b_ref)

    qf3 = H1f_ref[...].reshape(LQ, B, H)
    qb3 = H1b_ref[...].reshape(LQ, B, H)
    wq = wq_ref[...]
    scores = (jnp.sum(qf3 * wq[0, :H].reshape(1, 1, H), axis=-1)
              + jnp.sum(qb3 * wq[0, H:].reshape(1, 1, H), axis=-1))
    maskq = jnp.swapaxes(mask_ref[...].astype(F32), 0, 1)
    scores = jnp.where(maskq > 0.0, -1e30, scores)
    m = jnp.max(scores, axis=0, keepdims=True)
    e = jnp.exp(scores - m)
    alpha = e / jnp.sum(e, axis=0, keepdims=True)
    qm_f = jnp.sum(alpha[:, :, None] * qf3, axis=0)
    qm_b = jnp.sum(alpha[:, :, None] * qb3, axis=0)
    vs_ref[...] = (jnp.dot(qm_f, WsT_ref[:H], preferred_element_type=F32)
                   + jnp.dot(qm_b, WsT_ref[H:], preferred_element_type=F32))
    ve_ref[...] = (jnp.dot(qm_f, WeT_ref[:H], preferred_element_type=F32)
                   + jnp.dot(qm_b, WeT_ref[H:], preferred_element_type=F32))


def _qblock(xq, maskq, W0f, Wh0f, b0f, W0b, Wh0b, b0b,
            W1ff, W1fb, Wh1f, b1f, W1bf, W1bb, Wh1b, b1b, wq, WsT, WeT):
    return pl.pallas_call(
        _q_body,
        out_shape=[
            jax.ShapeDtypeStruct((B, 2 * H), F32),
            jax.ShapeDtypeStruct((B, 2 * H), F32),
        ],
        scratch_shapes=[
            pltpu.VMEM((LQ * B, 4 * H), F32),
            pltpu.VMEM((LQ * B, 4 * H), F32),
            pltpu.VMEM((LQ * B, H), F32),
            pltpu.VMEM((LQ * B, H), F32),
            pltpu.VMEM((LQ * B, H), F32),
            pltpu.VMEM((LQ * B, H), F32),
        ],
    )(xq, maskq, W0f, Wh0f, b0f, W0b, Wh0b, b0b,
      W1ff, W1fb, Wh1f, b1f, W1bf, W1bb, Wh1b, b1b, wq, WsT, WeT)


def _final_body(hf_ref, hb_ref, vs_ref, ve_ref, mask_ref, out_ref, S_ref, E_ref):
    vs = vs_ref[...]
    ve = ve_ref[...]
    vsf = vs[:, :H].reshape(1, B, H)
    vsb = vs[:, H:].reshape(1, B, H)
    vef = ve[:, :H].reshape(1, B, H)
    veb = ve[:, H:].reshape(1, B, H)
    CH = 64
    for c in range(LD // CH):
        sl = pl.ds(c * CH, CH)
        hf = hf_ref[sl]
        hb = hb_ref[sl]
        S_ref[sl] = jnp.sum(hf * vsf, axis=-1) + jnp.sum(hb * vsb, axis=-1)
        E_ref[sl] = jnp.sum(hf * vef, axis=-1) + jnp.sum(hb * veb, axis=-1)

    mask = mask_ref[...]

    def logsm(x_tb):
        x = jnp.where(mask, -1e30, jnp.swapaxes(x_tb, 0, 1))
        m = jnp.max(x, axis=1, keepdims=True)
        return x - m - jnp.log(jnp.sum(jnp.exp(x - m), axis=1, keepdims=True))

    out_ref[0] = logsm(S_ref[...])
    out_ref[1] = logsm(E_ref[...])


def _final(hf1, hb1, vs, ve, maskd):
    return pl.pallas_call(
        _final_body,
        out_shape=jax.ShapeDtypeStruct((2, B, LD), F32),
        scratch_shapes=[
            pltpu.VMEM((LD, B), F32),
            pltpu.VMEM((LD, B), F32),
        ],
    )(hf1, hb1, vs, ve, maskd)


def kernel(x1, x1_f, x1_mask, x2, x2_mask, emb,
           d0_Wxf, d0_Whf, d0_bf, d0_Wxb, d0_Whb, d0_bb,
           d1_Wxf, d1_Whf, d1_bf, d1_Wxb, d1_Whb, d1_bb,
           q0_Wxf, q0_Whf, q0_bf, q0_Wxb, q0_Whb, q0_bb,
           q1_Wxf, q1_Whf, q1_bf, q1_Wxb, q1_Whb, q1_bb,
           wq, Ws, We):
    embp = _transpose_pad(jnp.swapaxes(emb, 0, 1))
    xd = _make_sc_gather(LD * B, DG)(
        embp, x1.T.reshape(-1).astype(jnp.int32)).reshape(LD, B, DG)
    xq = _make_sc_gather(LQ * B, DG)(
        embp, x2.T.reshape(-1).astype(jnp.int32)).reshape(LQ, B, DG)
    fd = jnp.transpose(x1_f, (1, 0, 2))

    b2 = lambda b: b.reshape(1, 4 * H)
    wpad = lambda W: jnp.pad(W[:DEMB], ((0, DG - DEMB), (0, 0)))
    hf0, hb0 = _layer(xd, fd,
                      wpad(d0_Wxf), d0_Wxf[DEMB:], d0_Whf, b2(d0_bf),
                      wpad(d0_Wxb), d0_Wxb[DEMB:], d0_Whb, b2(d0_bb))
    hf1, hb1 = _layer(hf0, hb0,
                      d1_Wxf[:H], d1_Wxf[H:], d1_Whf, b2(d1_bf),
                      d1_Wxb[:H], d1_Wxb[H:], d1_Whb, b2(d1_bb))

    vs, ve = _qblock(xq, x2_mask,
                     wpad(q0_Wxf), q0_Whf, b2(q0_bf),
                     wpad(q0_Wxb), q0_Whb, b2(q0_bb),
                     q1_Wxf[:H], q1_Wxf[H:], q1_Whf, b2(q1_bf),
                     q1_Wxb[:H], q1_Wxb[H:], q1_Whb, b2(q1_bb),
                     wq.reshape(1, 2 * H), Ws.T, We.T)

    return _final(hf1, hb1, vs, ve, x1_mask)


# speedup vs baseline: 1.0598x; 1.0265x over previous
import functools

import jax
import jax.numpy as jnp
from jax import lax
from jax.experimental import pallas as pl
from jax.experimental.pallas import tpu as pltpu
from jax.experimental.pallas import tpu_sc as plsc

F32 = jnp.float32
H = 128
B = 32
LD = 512
LQ = 32
DEMB = 300
DG = 384


def _sig(x):
    return 0.5 * jnp.tanh(0.5 * x) + 0.5


def _make_sc_gather(n_rows, d):
    info = plsc.get_sparse_core_info()
    nw = info.num_cores * info.num_subcores
    per_w = n_rows // nw
    chunk = per_w
    while chunk * d * 4 > 300_000 or per_w % chunk:
        chunk -= 8
    n_ch = per_w // chunk
    mesh = plsc.VectorSubcoreMesh(core_axis_name="c", subcore_axis_name="s")

    @functools.partial(
        pl.kernel,
        mesh=mesh,
        out_type=jax.ShapeDtypeStruct((n_rows, d), F32),
        scratch_types=[
            pltpu.VMEM((chunk,), jnp.int32),
            pltpu.VMEM((chunk, d), F32),
            pltpu.SemaphoreType.DMA,
            pltpu.VMEM((chunk,), jnp.int32),
            pltpu.VMEM((chunk, d), F32),
            pltpu.SemaphoreType.DMA,
        ],
    )
    def k(table_hbm, idx_hbm, out_hbm, idx0, rows0, sem0, idx1, rows1, sem1):
        wid = lax.axis_index("s") * info.num_cores + lax.axis_index("c")
        base = wid * per_w
        bufs = [(idx0, rows0, sem0), (idx1, rows1, sem1)]
        pend = [None, None]

        def start(c):
            ib, rb, sm = bufs[c % 2]
            pltpu.sync_copy(idx_hbm.at[pl.ds(base + c * chunk, chunk)], ib)
            pend[c % 2] = pltpu.async_copy(table_hbm.at[ib], rb, sm)

        start(0)
        for c in range(n_ch):
            if c + 1 < n_ch:
                start(c + 1)
            pend[c % 2].wait()
            pltpu.sync_copy(bufs[c % 2][1],
                            out_hbm.at[pl.ds(base + c * chunk, chunk)])

    return k


_TCH = 2048


def _tpad_body(x_ref, o_ref):
    xT = jnp.swapaxes(x_ref[...], 0, 1)
    o_ref[...] = jnp.concatenate(
        [xT, jnp.zeros((_TCH, DG - DEMB), F32)], axis=1)


def _transpose_pad(embT):
    nblk = (embT.shape[1] + _TCH - 1) // _TCH
    return pl.pallas_call(
        _tpad_body,
        grid=(nblk,),
        in_specs=[pl.BlockSpec((DEMB, _TCH), lambda i: (0, i))],
        out_specs=pl.BlockSpec((_TCH, DG), lambda i: (i, 0)),
        out_shape=jax.ShapeDtypeStruct((nblk * _TCH, DG), F32),
    )(embT)


def _layer_body(x1f_ref, x2f_ref, x1b_ref, x2b_ref,
                W1f_ref, W2f_ref, Whf_ref, bf_ref,
                W1b_ref, W2b_ref, Whb_ref, bb_ref,
                outf_ref, outb_ref,
                Gf_ref, Gb_ref, hf_ref, cf_ref, hb_ref, cb_ref,
                *, ch, nch):
    i = pl.program_id(0)
    d1 = x1f_ref.shape[2]
    d2 = x2f_ref.shape[2]

    X1f = x1f_ref[...].reshape(ch * B, d1)
    X2f = x2f_ref[...].reshape(ch * B, d2)
    X1b = x1b_ref[...].reshape(ch * B, d1)
    X2b = x2b_ref[...].reshape(ch * B, d2)
    W1f, W2f = W1f_ref[...], W2f_ref[...]
    W1b, W2b = W1b_ref[...], W2b_ref[...]
    bfv, bbv = bf_ref[...], bb_ref[...]

    @pl.when(i == 0)
    def _init():
        z = jnp.zeros((B, H), F32)
        hf_ref[...] = z
        cf_ref[...] = z
        hb_ref[...] = z
        cb_ref[...] = z

    hf, cf = hf_ref[...], cf_ref[...]
    hb, cb = hb_ref[...], cb_ref[...]
    Whf, Whb = Whf_ref[...], Whb_ref[...]

    sub = 1
    NP = ch // sub
    for k in range(NP):
        lo = k * sub * B
        Gf_ref[lo:lo + sub * B, :] = (
            jnp.dot(X1f[lo:lo + sub * B], W1f, preferred_element_type=F32)
            + jnp.dot(X2f[lo:lo + sub * B], W2f, preferred_element_type=F32)
            + bfv)
        blo = (NP - 1 - k) * sub * B
        Gb_ref[blo:blo + sub * B, :] = (
            jnp.dot(X1b[blo:blo + sub * B], W1b, preferred_element_type=F32)
            + jnp.dot(X2b[blo:blo + sub * B], W2b, preferred_element_type=F32)
            + bbv)
        for t in range(k * sub, (k + 1) * sub):
            gf = Gf_ref[t * B:(t + 1) * B, :] + jnp.dot(
                hf, Whf, preferred_element_type=F32)
            tb = ch - 1 - t
            gb = Gb_ref[tb * B:(tb + 1) * B, :] + jnp.dot(
                hb, Whb, preferred_element_type=F32)
            cf = _sig(gf[:, H:2 * H]) * cf + _sig(gf[:, :H]) * jnp.tanh(
                gf[:, 2 * H:3 * H])
            hf = _sig(gf[:, 3 * H:]) * jnp.tanh(cf)
            outf_ref[t] = hf
            cb = _sig(gb[:, H:2 * H]) * cb + _sig(gb[:, :H]) * jnp.tanh(
                gb[:, 2 * H:3 * H])
            hb = _sig(gb[:, 3 * H:]) * jnp.tanh(cb)
            outb_ref[tb] = hb

    hf_ref[...] = hf
    cf_ref[...] = cf
    hb_ref[...] = hb
    cb_ref[...] = cb


def _layer(x1, x2, W1f, W2f, Whf, bf, W1b, W2b, Whb, bb, ch=64):
    T = x1.shape[0]
    d1, d2 = x1.shape[2], x2.shape[2]
    nch = T // ch
    wspec = lambda shape: pl.BlockSpec(shape, lambda i: (0,) * len(shape))
    return pl.pallas_call(
        functools.partial(_layer_body, ch=ch, nch=nch),
        grid=(nch,),
        in_specs=[
            pl.BlockSpec((ch, B, d1), lambda i: (i, 0, 0)),
            pl.BlockSpec((ch, B, d2), lambda i: (i, 0, 0)),
            pl.BlockSpec((ch, B, d1), lambda i, n=nch: (n - 1 - i, 0, 0)),
            pl.BlockSpec((ch, B, d2), lambda i, n=nch: (n - 1 - i, 0, 0)),
            wspec((d1, 4 * H)), wspec((d2, 4 * H)), wspec((H, 4 * H)),
            wspec((1, 4 * H)),
            wspec((d1, 4 * H)), wspec((d2, 4 * H)), wspec((H, 4 * H)),
            wspec((1, 4 * H)),
        ],
        out_specs=[
            pl.BlockSpec((ch, B, H), lambda i: (i, 0, 0)),
            pl.BlockSpec((ch, B, H), lambda i, n=nch: (n - 1 - i, 0, 0)),
        ],
        out_shape=[
            jax.ShapeDtypeStruct((T, B, H), F32),
            jax.ShapeDtypeStruct((T, B, H), F32),
        ],
        scratch_shapes=[
            pltpu.VMEM((ch * B, 4 * H), F32),
            pltpu.VMEM((ch * B, 4 * H), F32),
            pltpu.VMEM((B, H), F32),
            pltpu.VMEM((B, H), F32),
            pltpu.VMEM((B, H), F32),
            pltpu.VMEM((B, H), F32),
        ],
    )(x1, x2, x1, x2, W1f, W2f, Whf, bf, W1b, W2b, Whb, bb)


def _q_body(xq_ref, mask_ref,
            W0f_ref, Wh0f_ref, b0f_ref, W0b_ref, Wh0b_ref, b0b_ref,
            W1ff_ref, W1fb_ref, Wh1f_ref, b1f_ref,
            W1bf_ref, W1bb_ref, Wh1b_ref, b1b_ref,
            wq_ref, WsT_ref, WeT_ref,
            vs_ref, ve_ref,
            Gf_ref, Gb_ref, H0f_ref, H0b_ref, H1f_ref, H1b_ref):
    X = xq_ref[...].reshape(LQ * B, DG)

    def birun(Whf, Whb, Hf_ref, Hb_ref):
        hf = jnp.zeros((B, H), F32)
        cf = jnp.zeros((B, H), F32)
        hb = jnp.zeros((B, H), F32)
        cb = jnp.zeros((B, H), F32)
        for t in range(LQ):
            tb = LQ - 1 - t
            gf = Gf_ref[t * B:(t + 1) * B, :] + jnp.dot(
                hf, Whf, preferred_element_type=F32)
            gb = Gb_ref[tb * B:(tb + 1) * B, :] + jnp.dot(
                hb, Whb, preferred_element_type=F32)
            cf = _sig(gf[:, H:2 * H]) * cf + _sig(gf[:, :H]) * jnp.tanh(
                gf[:, 2 * H:3 * H])
            hf = _sig(gf[:, 3 * H:]) * jnp.tanh(cf)
            Hf_ref[t * B:(t + 1) * B, :] = hf
            cb = _sig(gb[:, H:2 * H]) * cb + _sig(gb[:, :H]) * jnp.tanh(
                gb[:, 2 * H:3 * H])
            hb = _sig(gb[:, 3 * H:]) * jnp.tanh(cb)
            Hb_ref[tb * B:(tb + 1) * B, :] = hb

    Gf_ref[...] = jnp.dot(X, W0f_ref[...], preferred_element_type=F32) + b0f_ref[...]
    Gb_ref[...] = jnp.dot(X, W0b_ref[...], preferred_element_type=F32) + b0b_ref[...]
    birun(Wh0f_ref[...], Wh0b_ref[...], H0f_ref, H0b_ref)
    h0f = H0f_ref[...]
    h0b = H0b_ref[...]
    Gf_ref[...] = (jnp.dot(h0f, W1ff_ref[...], preferred_element_type=F32)
                   + jnp.dot(h0b, W1fb_ref[...], preferred_element_type=F32)
                   + b1f_ref[...])
    Gb_ref[...] = (jnp.dot(h0f, W1bf_ref[...], preferred_element_type=F32)
                   + jnp.dot(h0b, W1bb_ref[...], preferred_element_type=F32)
                   + b1b_ref[...])
    birun(Wh1f_ref[...], Wh1b_ref[...], H1f_ref, H1b_ref)

    qf3 = H1f_ref[...].reshape(LQ, B, H)
    qb3 = H1b_ref[...].reshape(LQ, B, H)
    wq = wq_ref[...]
    scores = (jnp.sum(qf3 * wq[0, :H].reshape(1, 1, H), axis=-1)
              + jnp.sum(qb3 * wq[0, H:].reshape(1, 1, H), axis=-1))
    maskq = jnp.swapaxes(mask_ref[...].astype(F32), 0, 1)
    scores = jnp.where(maskq > 0.0, -1e30, scores)
    m = jnp.max(scores, axis=0, keepdims=True)
    e = jnp.exp(scores - m)
    alpha = e / jnp.sum(e, axis=0, keepdims=True)
    qm_f = jnp.sum(alpha[:, :, None] * qf3, axis=0)
    qm_b = jnp.sum(alpha[:, :, None] * qb3, axis=0)
    vs_ref[...] = (jnp.dot(qm_f, WsT_ref[:H], preferred_element_type=F32)
                   + jnp.dot(qm_b, WsT_ref[H:], preferred_element_type=F32))
    ve_ref[...] = (jnp.dot(qm_f, WeT_ref[:H], preferred_element_type=F32)
                   + jnp.dot(qm_b, WeT_ref[H:], preferred_element_type=F32))


def _qblock(xq, maskq, W0f, Wh0f, b0f, W0b, Wh0b, b0b,
            W1ff, W1fb, Wh1f, b1f, W1bf, W1bb, Wh1b, b1b, wq, WsT, WeT):
    return pl.pallas_call(
        _q_body,
        out_shape=[
            jax.ShapeDtypeStruct((B, 2 * H), F32),
            jax.ShapeDtypeStruct((B, 2 * H), F32),
        ],
        scratch_shapes=[
            pltpu.VMEM((LQ * B, 4 * H), F32),
            pltpu.VMEM((LQ * B, 4 * H), F32),
            pltpu.VMEM((LQ * B, H), F32),
            pltpu.VMEM((LQ * B, H), F32),
            pltpu.VMEM((LQ * B, H), F32),
            pltpu.VMEM((LQ * B, H), F32),
        ],
    )(xq, maskq, W0f, Wh0f, b0f, W0b, Wh0b, b0b,
      W1ff, W1fb, Wh1f, b1f, W1bf, W1bb, Wh1b, b1b, wq, WsT, WeT)


def _final_body(hf_ref, hb_ref, vs_ref, ve_ref, mask_ref, out_ref, S_ref, E_ref):
    vs = vs_ref[...]
    ve = ve_ref[...]
    vsf = vs[:, :H].reshape(1, B, H)
    vsb = vs[:, H:].reshape(1, B, H)
    vef = ve[:, :H].reshape(1, B, H)
    veb = ve[:, H:].reshape(1, B, H)
    CH = 64
    for c in range(LD // CH):
        sl = pl.ds(c * CH, CH)
        hf = hf_ref[sl]
        hb = hb_ref[sl]
        S_ref[sl] = jnp.sum(hf * vsf, axis=-1) + jnp.sum(hb * vsb, axis=-1)
        E_ref[sl] = jnp.sum(hf * vef, axis=-1) + jnp.sum(hb * veb, axis=-1)

    mask = mask_ref[...]

    def logsm(x_tb):
        x = jnp.where(mask, -1e30, jnp.swapaxes(x_tb, 0, 1))
        m = jnp.max(x, axis=1, keepdims=True)
        return x - m - jnp.log(jnp.sum(jnp.exp(x - m), axis=1, keepdims=True))

    out_ref[0] = logsm(S_ref[...])
    out_ref[1] = logsm(E_ref[...])


def _final(hf1, hb1, vs, ve, maskd):
    return pl.pallas_call(
        _final_body,
        out_shape=jax.ShapeDtypeStruct((2, B, LD), F32),
        scratch_shapes=[
            pltpu.VMEM((LD, B), F32),
            pltpu.VMEM((LD, B), F32),
        ],
    )(hf1, hb1, vs, ve, maskd)


def kernel(x1, x1_f, x1_mask, x2, x2_mask, emb,
           d0_Wxf, d0_Whf, d0_bf, d0_Wxb, d0_Whb, d0_bb,
           d1_Wxf, d1_Whf, d1_bf, d1_Wxb, d1_Whb, d1_bb,
           q0_Wxf, q0_Whf, q0_bf, q0_Wxb, q0_Whb, q0_bb,
           q1_Wxf, q1_Whf, q1_bf, q1_Wxb, q1_Whb, q1_bb,
           wq, Ws, We):
    embp = _transpose_pad(jnp.swapaxes(emb, 0, 1))
    xd = _make_sc_gather(LD * B, DG)(
        embp, x1.T.reshape(-1).astype(jnp.int32)).reshape(LD, B, DG)
    xq = _make_sc_gather(LQ * B, DG)(
        embp, x2.T.reshape(-1).astype(jnp.int32)).reshape(LQ, B, DG)
    fd = jnp.transpose(x1_f, (1, 0, 2))

    b2 = lambda b: b.reshape(1, 4 * H)
    wpad = lambda W: jnp.pad(W[:DEMB], ((0, DG - DEMB), (0, 0)))
    hf0, hb0 = _layer(xd, fd,
                      wpad(d0_Wxf), d0_Wxf[DEMB:], d0_Whf, b2(d0_bf),
                      wpad(d0_Wxb), d0_Wxb[DEMB:], d0_Whb, b2(d0_bb))
    hf1, hb1 = _layer(hf0, hb0,
                      d1_Wxf[:H], d1_Wxf[H:], d1_Whf, b2(d1_bf),
                      d1_Wxb[:H], d1_Wxb[H:], d1_Whb, b2(d1_bb))

    vs, ve = _qblock(xq, x2_mask,
                     wpad(q0_Wxf), q0_Whf, b2(q0_bf),
                     wpad(q0_Wxb), q0_Whb, b2(q0_bb),
                     q1_Wxf[:H], q1_Wxf[H:], q1_Whf, b2(q1_bf),
                     q1_Wxb[:H], q1_Wxb[H:], q1_Whb, b2(q1_bb),
                     wq.reshape(1, 2 * H), Ws.T, We.T)

    return _final(hf1, hb1, vs, ve, x1_mask)
